# baseline (speedup 1.0000x reference)
import math

import jax
import jax.numpy as jnp
from jax.experimental import pallas as pl
from jax.experimental.pallas import tpu as pltpu

L = 200
K = 16
D = 32
G = 4
TAIL0 = L - K
WSUM = float(sum(math.exp(i - L) for i in range(L)))
WT = [math.exp(t - K) for t in range(K)]


def _gae_kernel(srcp_ref, dstp_ref, tl_ref, e_ref, sel_ref, w1_ref, b1_ref,
                w2d_ref, b2_ref, so_ref, do_ref):
    srcp = srcp_ref[...]
    dstp = dstp_ref[...]
    tl = tl_ref[...]
    e = e_ref[...]
    rp = srcp.shape[0]

    acc_s = jnp.zeros((rp, 2 * K * G), jnp.float32)
    acc_d = jnp.zeros((rp, 2 * K * G), jnp.float32)
    for j in range(L):
        cmp_s = jnp.dot(srcp[:, G * j:G * j + G], e,
                        preferred_element_type=jnp.float32,
                        precision=jax.lax.Precision.HIGHEST)
        acc_s = acc_s + (tl == cmp_s).astype(jnp.float32)
        cmp_d = jnp.dot(dstp[:, G * j:G * j + G], e,
                        preferred_element_type=jnp.float32,
                        precision=jax.lax.Precision.HIGHEST)
        acc_d = acc_d + (tl == cmp_d).astype(jnp.float32)
    valid = tl != 0.0
    cs = jnp.where(valid, acc_s, 0.0)
    cd = jnp.where(valid, acc_d, 0.0)

    sel = sel_ref[...]
    ex_s = jnp.dot(cs, sel, preferred_element_type=jnp.float32)
    ex_d = jnp.dot(cd, sel, preferred_element_type=jnp.float32)

    w1 = w1_ref[...]
    b1 = b1_ref[...]
    accf_s = jnp.zeros((rp, 2 * K * G), jnp.float32)
    accf_d = jnp.zeros((rp, 2 * K * G), jnp.float32)
    for t in range(K):
        wt = WT[t]
        bs = jnp.dot(ex_s[:, G * t:G * t + G], e,
                     preferred_element_type=jnp.float32)
        bd = jnp.dot(ex_d[:, G * t:G * t + G], e,
                     preferred_element_type=jnp.float32)
        accf_s = accf_s + wt * (jax.nn.relu(bs * w1 + b1)
                                + jax.nn.relu(bd * w1 + b1))
        t2 = K + t
        bs2 = jnp.dot(ex_s[:, G * t2:G * t2 + G], e,
                      preferred_element_type=jnp.float32)
        bd2 = jnp.dot(ex_d[:, G * t2:G * t2 + G], e,
                      preferred_element_type=jnp.float32)
        accf_d = accf_d + wt * (jax.nn.relu(bs2 * w1 + b1)
                                + jax.nn.relu(bd2 * w1 + b1))

    w2d = w2d_ref[...]
    b2 = b2_ref[...]
    inv = 1.0 / WSUM
    so_ref[...] = (jnp.dot(accf_s, w2d, preferred_element_type=jnp.float32)
                   * inv + b2)
    do_ref[...] = (jnp.dot(accf_d, w2d, preferred_element_type=jnp.float32)
                   * inv + b2)


def kernel(src_padded_nodes_neighbor_ids, dst_padded_nodes_neighbor_ids,
           W1, b1, W2, b2):
    src = src_padded_nodes_neighbor_ids
    dst = dst_padded_nodes_neighbor_ids
    B = src.shape[0]
    Bp = B // G
    f32 = jnp.float32

    srcf = src.astype(f32)
    dstf = dst.astype(f32)
    srcp = srcf.reshape(Bp, G, L).transpose(0, 2, 1).reshape(Bp, G * L)
    dstp = dstf.reshape(Bp, G, L).transpose(0, 2, 1).reshape(Bp, G * L)
    tl = jnp.concatenate([srcf[:, TAIL0:], dstf[:, TAIL0:]],
                         axis=1).reshape(Bp, G * 2 * K)

    lanes = jnp.arange(128)
    e = (lanes[None, :] // (2 * K) == jnp.arange(G)[:, None]).astype(f32)
    sel = (lanes[:, None] ==
           (2 * K) * (lanes[None, :] % G) + lanes[None, :] // G).astype(f32)

    w1r = jnp.tile(W1.reshape(1, D), (1, G))
    b1r = jnp.tile(b1.reshape(1, D), (1, G))
    b2r = jnp.tile((2.0 * b2).reshape(1, D), (1, G))
    w2d = jnp.kron(jnp.eye(G, dtype=f32), W2.T)

    Rp = 64
    grid = (Bp // Rp,)
    so, do = pl.pallas_call(
        _gae_kernel,
        grid=grid,
        in_specs=[
            pl.BlockSpec((Rp, G * L), lambda i: (i, 0)),
            pl.BlockSpec((Rp, G * L), lambda i: (i, 0)),
            pl.BlockSpec((Rp, 128), lambda i: (i, 0)),
            pl.BlockSpec((G, 128), lambda i: (0, 0)),
            pl.BlockSpec((128, 128), lambda i: (0, 0)),
            pl.BlockSpec((1, 128), lambda i: (0, 0)),
            pl.BlockSpec((1, 128), lambda i: (0, 0)),
            pl.BlockSpec((128, 128), lambda i: (0, 0)),
            pl.BlockSpec((1, 128), lambda i: (0, 0)),
        ],
        out_specs=[pl.BlockSpec((Rp, 128), lambda i: (i, 0)),
                   pl.BlockSpec((Rp, 128), lambda i: (i, 0))],
        out_shape=[jax.ShapeDtypeStruct((Bp, 128), f32),
                   jax.ShapeDtypeStruct((Bp, 128), f32)],
        compiler_params=pltpu.CompilerParams(
            dimension_semantics=("parallel",)),
    )(srcp, dstp, tl, e, sel, w1r, b1r, w2d, b2r)
    return (so.reshape(B, D), do.reshape(B, D))


# speedup vs baseline: 4.1913x; 4.1913x over previous
import math

import jax
import jax.numpy as jnp
from jax.experimental import pallas as pl
from jax.experimental.pallas import tpu as pltpu

L = 200
K = 16
D = 32
TAIL0 = L - K
T2 = 2 * K
WSUM = float(sum(math.exp(i - L) for i in range(L)))
WT = [math.exp(t - K) for t in range(K)]


def _gae_kernel(srcT_ref, dstT_ref, tlT_ref, w1b_ref, b1b_ref, w2_ref,
                b2b_ref, so_ref, do_ref):
    src = srcT_ref[...]
    dst = dstT_ref[...]
    tl = tlT_ref[...]
    rb = src.shape[1]

    acc_s = jnp.zeros((T2, rb), jnp.int32)
    acc_d = jnp.zeros((T2, rb), jnp.int32)
    for j in range(L):
        cj_s = jnp.broadcast_to(src[j:j + 1, :], (T2, rb))
        acc_s = acc_s + (tl == cj_s).astype(jnp.int32)
        cj_d = jnp.broadcast_to(dst[j:j + 1, :], (T2, rb))
        acc_d = acc_d + (tl == cj_d).astype(jnp.int32)
    valid = tl != 0
    cs = jnp.where(valid, acc_s, 0).astype(jnp.float32)
    cd = jnp.where(valid, acc_d, 0).astype(jnp.float32)

    w1b = w1b_ref[...]
    b1b = b1b_ref[...]
    accf_s = jnp.zeros((D, rb), jnp.float32)
    accf_d = jnp.zeros((D, rb), jnp.float32)
    for t in range(K):
        wt = WT[t]
        c1 = jnp.broadcast_to(cs[t:t + 1, :], (D, rb))
        c2 = jnp.broadcast_to(cd[t:t + 1, :], (D, rb))
        accf_s = accf_s + wt * (jax.nn.relu(c1 * w1b + b1b)
                                + jax.nn.relu(c2 * w1b + b1b))
        c1d = jnp.broadcast_to(cs[K + t:K + t + 1, :], (D, rb))
        c2d = jnp.broadcast_to(cd[K + t:K + t + 1, :], (D, rb))
        accf_d = accf_d + wt * (jax.nn.relu(c1d * w1b + b1b)
                                + jax.nn.relu(c2d * w1b + b1b))

    w2 = w2_ref[...]
    b2b = b2b_ref[...]
    inv = 1.0 / WSUM
    so_ref[...] = (jnp.dot(w2, accf_s, preferred_element_type=jnp.float32)
                   * inv + b2b)
    do_ref[...] = (jnp.dot(w2, accf_d, preferred_element_type=jnp.float32)
                   * inv + b2b)


def kernel(src_padded_nodes_neighbor_ids, dst_padded_nodes_neighbor_ids,
           W1, b1, W2, b2):
    src = src_padded_nodes_neighbor_ids
    dst = dst_padded_nodes_neighbor_ids
    B = src.shape[0]
    f32 = jnp.float32

    srcT = src.T
    dstT = dst.T
    tlT = jnp.concatenate([src[:, TAIL0:], dst[:, TAIL0:]], axis=1).T

    ones = jnp.ones((1, B), f32)
    w1b = W1.reshape(D, 1) * ones
    b1b = b1.reshape(D, 1) * ones
    b2b = (2.0 * b2).reshape(D, 1) * ones

    Rb = 128
    grid = (B // Rb,)
    so, do = pl.pallas_call(
        _gae_kernel,
        grid=grid,
        in_specs=[
            pl.BlockSpec((L, Rb), lambda i: (0, i)),
            pl.BlockSpec((L, Rb), lambda i: (0, i)),
            pl.BlockSpec((T2, Rb), lambda i: (0, i)),
            pl.BlockSpec((D, Rb), lambda i: (0, i)),
            pl.BlockSpec((D, Rb), lambda i: (0, i)),
            pl.BlockSpec((D, D), lambda i: (0, 0)),
            pl.BlockSpec((D, Rb), lambda i: (0, i)),
        ],
        out_specs=[pl.BlockSpec((D, Rb), lambda i: (0, i)),
                   pl.BlockSpec((D, Rb), lambda i: (0, i))],
        out_shape=[jax.ShapeDtypeStruct((D, B), f32),
                   jax.ShapeDtypeStruct((D, B), f32)],
        compiler_params=pltpu.CompilerParams(
            dimension_semantics=("parallel",)),
    )(srcT, dstT, tlT, w1b, b1b, W2, b2b)
    return (so.T, do.T)
